# Initial kernel scaffold; baseline (speedup 1.0000x reference)
import jax
import jax.numpy as jnp
from jax.experimental import pallas as pl


def kernel(chars, char_counts, words, word_counts, char_emb, word_emb, w_ih_f, w_hh_f, b_ih_f, b_hh_f, w_ih_b, w_hh_b, b_ih_b, b_hh_b, w_out, b_out):
    raise NotImplementedError("write your pallas kernel here")


# speedup vs baseline: 30.8404x; 30.8404x over previous
import functools

import jax
import jax.numpy as jnp
from jax import lax
from jax.experimental import pallas as pl
from jax.experimental.pallas import tpu as pltpu
from jax.experimental.pallas import tpu_sc as plsc

H = 8
CE = 16
WE = 32
NCHARS = 128
NLABELS = 64
C = 16
_BLK = 1024


def _pack_weights(char_emb, w_ih_f, w_hh_f, b_ih_f, b_hh_f,
                  w_ih_b, w_hh_b, b_ih_b, b_hh_b, w_out, b_out):
    f32 = jnp.float32
    ce = char_emb.astype(f32).at[0].set(0.0)[:NCHARS]

    def perm(mf, mb):
        cols = []
        for g in range(4):
            cols.append(mf[:, g * H:(g + 1) * H])
            cols.append(mb[:, g * H:(g + 1) * H])
        return jnp.concatenate(cols, axis=1)

    wih = perm(w_ih_f.T, w_ih_b.T)
    bias = perm((b_ih_f + b_hh_f)[None, :],
                (b_ih_b + b_hh_b)[None, :])
    afull = ce @ wih + bias
    col = jnp.arange(8 * H)
    fwdcols = ((col // H) % 2 == 0)[None, :]
    a_top = jnp.where(fwdcols, afull, 0.0)
    a_bot = jnp.where(fwdcols, 0.0, afull)
    a2 = jnp.concatenate([a_top, a_bot], axis=0)
    z = jnp.zeros((H, 4 * H), f32)
    whh = perm(jnp.concatenate([w_hh_f.T, z], axis=0),
               jnp.concatenate([z, w_hh_b.T], axis=0))
    wot = w_out.T
    woc = wot[WE:]
    wow = wot[:WE]
    bout = b_out[None, :]
    return a2, whh, woc, wow, bout


def _word_gather(word_emb, words_flat, n):
    info = plsc.get_sparse_core_info()
    nw = info.num_cores * info.num_subcores
    chunk = 128
    b_per_w = n // nw
    n_chunks = b_per_w // chunk
    idx3 = words_flat.reshape(nw, n_chunks, chunk)
    mesh = plsc.VectorSubcoreMesh(core_axis_name="c", subcore_axis_name="s")

    @functools.partial(
        pl.kernel, mesh=mesh,
        out_type=jax.ShapeDtypeStruct((n, WE), jnp.float32),
        compiler_params=pltpu.CompilerParams(use_tc_tiling_on_sc=False),
        scratch_types=[
            pltpu.VMEM((n_chunks, chunk), jnp.int32),
            pltpu.VMEM((b_per_w, WE), jnp.float32),
            pltpu.SemaphoreType.DMA,
        ],
    )
    def gather_k(table_hbm, idx_hbm, out_hbm, idx_v, rows_v, sem):
        wid = lax.axis_index("s") * info.num_cores + lax.axis_index("c")
        pltpu.sync_copy(idx_hbm.at[wid], idx_v)
        copies = [
            pltpu.async_copy(table_hbm.at[idx_v.at[j]],
                             rows_v.at[pl.ds(j * chunk, chunk)], sem)
            for j in range(n_chunks)
        ]
        for cp in copies:
            cp.wait()
        pltpu.sync_copy(rows_v, out_hbm.at[pl.ds(wid * b_per_w, b_per_w)])

    return gather_k(word_emb, idx3)


def _tc_body(chars_ref, lens_ref, words_ref, wrep_ref,
             a2_ref, whh_ref, woc_ref, wow_ref, bout_ref, out_ref):
    B = chars_ref.shape[0]
    f32 = jnp.float32
    idx256 = lax.broadcasted_iota(jnp.int32, (B, 2 * NCHARS), 1)
    a2 = a2_ref[...]
    gx = []
    for k in range(C):
        ck = chars_ref[:, k:k + 1]
        cr = chars_ref[:, C - 1 - k:C - k] + NCHARS
        ohc = jnp.logical_or(idx256 == ck, idx256 == cr).astype(f32)
        gx.append(jnp.dot(ohc, a2, preferred_element_type=f32))

    lens = lens_ref[...]
    col16 = lax.broadcasted_iota(jnp.int32, (B, 2 * H), 1)
    col64 = lax.broadcasted_iota(jnp.int32, (1, 8 * H), 1)
    gate_scale = jnp.where((col64 >= 4 * H) & (col64 < 6 * H),
                           1.0, 0.5).astype(f32)
    whh = whh_ref[...]
    h = jnp.zeros((B, 2 * H), f32)
    c = jnp.zeros((B, 2 * H), f32)
    acc = jnp.zeros((B, 2 * H), f32)
    for k in range(C):
        gates = gx[k] + jnp.dot(h, whh, preferred_element_type=f32)
        th = jnp.tanh(gates * gate_scale)
        i_g = 0.5 * th[:, 0:2 * H] + 0.5
        f_g = 0.5 * th[:, 2 * H:4 * H] + 0.5
        g_g = th[:, 4 * H:6 * H]
        o_g = 0.5 * th[:, 6 * H:8 * H] + 0.5
        c_new = f_g * c + i_g * g_g
        h_new = o_g * jnp.tanh(c_new)
        tsel = jnp.where(col16 < H, k, C - 1 - k)
        m = tsel < lens
        h = jnp.where(m, h_new, h)
        c = jnp.where(m, c_new, c)
        acc = acc + jnp.where(m, h_new, 0.0)

    wmask = words_ref[...] != 0
    wrep = jnp.where(wmask, wrep_ref[...], 0.0)
    out_ref[...] = (jnp.dot(acc, woc_ref[...], preferred_element_type=f32)
                    + jnp.dot(wrep, wow_ref[...], preferred_element_type=f32)
                    + bout_ref[...])


def _tc_call(chars2, lens2, words2, wrep, a2, whh, woc, wow, bout, n):
    B = _BLK
    return pl.pallas_call(
        _tc_body,
        grid=(n // B,),
        in_specs=[
            pl.BlockSpec((B, C), lambda i: (i, 0)),
            pl.BlockSpec((B, 1), lambda i: (i, 0)),
            pl.BlockSpec((B, 1), lambda i: (i, 0)),
            pl.BlockSpec((B, WE), lambda i: (i, 0)),
            pl.BlockSpec((2 * NCHARS, 8 * H), lambda i: (0, 0)),
            pl.BlockSpec((2 * H, 8 * H), lambda i: (0, 0)),
            pl.BlockSpec((2 * H, NLABELS), lambda i: (0, 0)),
            pl.BlockSpec((WE, NLABELS), lambda i: (0, 0)),
            pl.BlockSpec((1, NLABELS), lambda i: (0, 0)),
        ],
        out_specs=pl.BlockSpec((B, NLABELS), lambda i: (i, 0)),
        out_shape=jax.ShapeDtypeStruct((n, NLABELS), jnp.float32),
    )(chars2, lens2, words2, wrep, a2, whh, woc, wow, bout)


def kernel(chars, char_counts, words, word_counts, char_emb, word_emb,
           w_ih_f, w_hh_f, b_ih_f, b_hh_f, w_ih_b, w_hh_b, b_ih_b, b_hh_b,
           w_out, b_out):
    Sd, Wd, Cd = chars.shape
    n = Sd * Wd
    chars2 = chars.reshape(n, Cd)
    lens2 = char_counts.reshape(n, 1)
    words_flat = words.reshape(n)
    words2 = words_flat.reshape(n, 1)
    a2, whh, woc, wow, bout = _pack_weights(
        char_emb, w_ih_f, w_hh_f, b_ih_f, b_hh_f,
        w_ih_b, w_hh_b, b_ih_b, b_hh_b, w_out, b_out)
    wrep = _word_gather(word_emb, words_flat, n)
    out = _tc_call(chars2, lens2, words2, wrep, a2, whh, woc, wow, bout, n)
    return out.reshape(Sd, Wd, NLABELS)
